# Initial kernel scaffold; baseline (speedup 1.0000x reference)
import jax
import jax.numpy as jnp
from jax.experimental import pallas as pl


def kernel(x, edge_index, batch, lin1_W, lin1_b, lin1_a, sg_W, sg_b, act1_W, act1_b, act1_a, gate_W1, gate_b1, gate_a, gate_W2, gate_b2, proj_W, proj_b):
    raise NotImplementedError("write your pallas kernel here")


# speedup vs baseline: 14.7255x; 14.7255x over previous
import functools

import jax
import jax.numpy as jnp
from jax import lax
from jax.experimental import pallas as pl
from jax.experimental.pallas import tpu as pltpu
from jax.experimental.pallas import tpu_sc as plsc

N = 10000
NPAD = 10240
HID = 128
HALF = 64
G = 8
NTILE = 16
RPT = NPAD // NTILE
E = 320000
ESEG = E // NTILE
CHUNK = 128
NCHUNK = 157
ESEGP = NCHUNK * CHUNK
BLK = 1000
F32 = jnp.float32

_MESH = dict(core_axis_name="c", subcore_axis_name="s", num_cores=2,
             num_subcores=NTILE)


def _deg_body(dst_hbm, dega_hbm, degb_hbm, dst_v, ones_v, zline_v, deg_sh,
              sem):
    c = lax.axis_index("c")
    s = lax.axis_index("s")
    pltpu.sync_copy(dst_hbm.at[s], dst_v)
    for j in range(CHUNK // 16):
        ones_v[pl.ds(j * 16, 16)] = jnp.full((16,), 1.0, F32)
    for j in range(RPT // 16):
        zline_v[pl.ds(j * 16, 16)] = jnp.zeros((16,), F32)
    pltpu.sync_copy(zline_v, deg_sh.at[pl.ds(s * RPT, RPT)])
    plsc.subcore_barrier()

    nk = 79 - c

    def body(i, _):
        k = 2 * i + c
        pltpu.async_copy(ones_v, deg_sh.at[dst_v.at[k]], sem, add=True).wait()
        return 0

    lax.fori_loop(0, nk, body, 0)
    plsc.subcore_barrier()

    @pl.when(c == 0)
    def _():
        pltpu.sync_copy(deg_sh.at[pl.ds(s * RPT, RPT)],
                        dega_hbm.at[pl.ds(s * RPT, RPT)])

    @pl.when(c == 1)
    def _():
        pltpu.sync_copy(deg_sh.at[pl.ds(s * RPT, RPT)],
                        degb_hbm.at[pl.ds(s * RPT, RPT)])


SUB = 160
NSUB = RPT // SUB


def _prop_body(g0l, g0r, srci, dsti, invd, ul, ur, t1l, t1r, t2l, t2r,
               esrc_v, edst_v, invd_v, slab_v, gbuf_v, acc_sh, sem):
    c = lax.axis_index("c")
    s = lax.axis_index("s")
    row0 = s * RPT

    pltpu.sync_copy(srci.at[s], esrc_v)
    pltpu.sync_copy(dsti.at[s], edst_v)
    pltpu.sync_copy(invd.at[pl.ds(row0, RPT)], invd_v)

    for cval, g0 in ((0, g0l), (1, g0r)):
        @pl.when(c == cval)
        def _(g0=g0):
            for sub in range(NSUB):
                r0 = row0 + sub * SUB
                pltpu.sync_copy(g0.at[pl.ds(r0, SUB)], slab_v)
                pltpu.sync_copy(slab_v, acc_sh.at[pl.ds(r0, SUB)])
    plsc.subcore_barrier()

    srcs = ((g0l, t1l, t2l), (g0r, t1r, t2r))

    for r in range(3):
        for cval in (0, 1):
            gsrc = srcs[cval][r]

            @pl.when(c == cval)
            def _(gsrc=gsrc):
                def ebody(k, _):
                    pltpu.async_copy(gsrc.at[esrc_v.at[k]], gbuf_v,
                                     sem).wait()
                    pltpu.async_copy(gbuf_v, acc_sh.at[edst_v.at[k]], sem,
                                     add=True).wait()
                    return 0

                lax.fori_loop(0, NCHUNK, ebody, 0)
        plsc.subcore_barrier()

        if r < 2:
            for sub in range(NSUB):
                r0 = row0 + sub * SUB
                pltpu.sync_copy(acc_sh.at[pl.ds(r0, SUB)], slab_v)

                def rbody(gi, _, sub=sub):
                    dv = invd_v[pl.ds(sub * SUB + gi * 16, 16)]
                    for rr in range(16):
                        d = dv[rr]
                        i = gi * 16 + rr
                        for j in range(HALF // 16):
                            sl = pl.ds(j * 16, 16)
                            slab_v[i, sl] = slab_v[i, sl] * d
                    return 0

                lax.fori_loop(0, SUB // 16, rbody, 0)
                pltpu.sync_copy(slab_v, acc_sh.at[pl.ds(r0, SUB)])
                for cval in (0, 1):
                    tnext = srcs[cval][r + 1]

                    @pl.when(c == cval)
                    def _(tnext=tnext):
                        pltpu.sync_copy(slab_v, tnext.at[pl.ds(r0, SUB)])
            plsc.subcore_barrier()
        else:
            for cval, uout in ((0, ul), (1, ur)):
                @pl.when(c == cval)
                def _(uout=uout):
                    pltpu.sync_copy(acc_sh.at[pl.ds(row0, RPT)],
                                    uout.at[pl.ds(row0, RPT)])


@functools.cache
def _sc_kernels():
    mesh = plsc.VectorSubcoreMesh(**_MESH)
    deg_k = pl.kernel(
        _deg_body,
        out_type=[jax.ShapeDtypeStruct((NPAD,), F32),
                  jax.ShapeDtypeStruct((NPAD,), F32)],
        mesh=mesh,
        scratch_types=[
            pltpu.VMEM((NCHUNK, CHUNK), jnp.int32),
            pltpu.VMEM((CHUNK,), F32),
            pltpu.VMEM((RPT,), F32),
            pltpu.VMEM_SHARED((NPAD,), F32),
            pltpu.SemaphoreType.DMA,
        ],
    )
    prop_k = pl.kernel(
        _prop_body,
        out_type=[jax.ShapeDtypeStruct((NPAD, HALF), F32)] * 6,
        mesh=mesh,
        scratch_types=[
            pltpu.VMEM((NCHUNK, CHUNK), jnp.int32),
            pltpu.VMEM((NCHUNK, CHUNK), jnp.int32),
            pltpu.VMEM((RPT,), F32),
            pltpu.VMEM((SUB, HALF), F32),
            pltpu.VMEM((CHUNK, HALF), F32),
            pltpu.VMEM_SHARED((NPAD, HALF), F32),
            pltpu.SemaphoreType.DMA,
        ],
        compiler_params=pltpu.CompilerParams(use_tc_tiling_on_sc=False),
    )
    return deg_k, prop_k


def _tc1_body(x_ref, w_ref, b_ref, a_ref, dega_ref, degb_ref,
              g0l_ref, g0r_ref, invd_ref, dinv_ref):
    h = jnp.dot(x_ref[...], w_ref[...], preferred_element_type=F32)
    h = h + b_ref[...]
    a = a_ref[0, 0]
    h = jnp.where(h >= 0, h, a * h)
    deg = dega_ref[...] + degb_ref[...] + 1.0
    dinv = lax.rsqrt(deg)
    g0 = h * dinv
    g0l_ref[...] = g0[:, :HALF]
    g0r_ref[...] = g0[:, HALF:]
    invd_ref[...] = 1.0 / deg
    dinv_ref[...] = dinv


def _tc1(x, w, b, a, dega, degb):
    return pl.pallas_call(
        _tc1_body,
        grid=(N // BLK,),
        in_specs=[
            pl.BlockSpec((BLK, HID), lambda i: (i, 0)),
            pl.BlockSpec((HID, HID), lambda i: (0, 0)),
            pl.BlockSpec((1, HID), lambda i: (0, 0)),
            pl.BlockSpec(memory_space=pltpu.SMEM),
            pl.BlockSpec((BLK, 1), lambda i: (i, 0)),
            pl.BlockSpec((BLK, 1), lambda i: (i, 0)),
        ],
        out_specs=[
            pl.BlockSpec((BLK, HALF), lambda i: (i, 0)),
            pl.BlockSpec((BLK, HALF), lambda i: (i, 0)),
            pl.BlockSpec((BLK, 1), lambda i: (i, 0)),
            pl.BlockSpec((BLK, 1), lambda i: (i, 0)),
        ],
        out_shape=[
            jax.ShapeDtypeStruct((NPAD, HALF), F32),
            jax.ShapeDtypeStruct((NPAD, HALF), F32),
            jax.ShapeDtypeStruct((NPAD, 1), F32),
            jax.ShapeDtypeStruct((NPAD, 1), F32),
        ],
    )(x, w, b, a, dega, degb)


def _tc2_body(ul_ref, ur_ref, dinv_ref, x_ref, sgw_ref, sgb_ref,
              a1w_ref, a1b_ref, a1a_ref, gw1_ref, gb1_ref, ga_ref,
              gw2_ref, gb2_ref, batch_ref, x1_ref, gate_ref, gmax_ref):
    i = pl.program_id(0)
    u = jnp.concatenate([ul_ref[...], ur_ref[...]], axis=1)
    h3 = u * dinv_ref[...]
    h = jnp.dot(h3, sgw_ref[...], preferred_element_type=F32) + sgb_ref[...]
    x1 = jnp.dot(h, a1w_ref[...], preferred_element_type=F32) + a1b_ref[...]
    a1 = a1a_ref[0, 0]
    x1 = jnp.where(x1 >= 0, x1, a1 * x1)
    x1_ref[...] = x1
    feats = jnp.concatenate([x_ref[...], x1], axis=1)
    g1 = jnp.dot(feats, gw1_ref[...], preferred_element_type=F32)
    g1 = g1 + gb1_ref[...]
    g1 = jnp.where(g1 >= 0, g1, ga_ref[...] * g1)
    gate = jnp.dot(g1, gw2_ref[...], preferred_element_type=F32)
    gate = gate + gb2_ref[0, 0]
    gate_ref[...] = gate

    onehot = batch_ref[...] == lax.broadcasted_iota(jnp.int32, (BLK, G), 1)
    gm = jnp.max(jnp.where(onehot, gate, jnp.float32(-3e38)), axis=0)[:, None]

    @pl.when(i == 0)
    def _():
        gmax_ref[...] = jnp.full((G, 1), -3e38, F32)

    gmax_ref[...] = jnp.maximum(gmax_ref[...], gm)


def _tc2(ul, ur, dinv, x, sgw, sgb, a1w, a1b, a1a, gw1, gb1, ga, gw2, gb2,
         batch2d):
    return pl.pallas_call(
        _tc2_body,
        grid=(N // BLK,),
        in_specs=[
            pl.BlockSpec((BLK, HALF), lambda i: (i, 0)),
            pl.BlockSpec((BLK, HALF), lambda i: (i, 0)),
            pl.BlockSpec((BLK, 1), lambda i: (i, 0)),
            pl.BlockSpec((BLK, HID), lambda i: (i, 0)),
            pl.BlockSpec((HID, 2 * HID), lambda i: (0, 0)),
            pl.BlockSpec((1, 2 * HID), lambda i: (0, 0)),
            pl.BlockSpec((2 * HID, HID), lambda i: (0, 0)),
            pl.BlockSpec((1, HID), lambda i: (0, 0)),
            pl.BlockSpec(memory_space=pltpu.SMEM),
            pl.BlockSpec((2 * HID, HID), lambda i: (0, 0)),
            pl.BlockSpec((1, HID), lambda i: (0, 0)),
            pl.BlockSpec((1, HID), lambda i: (0, 0)),
            pl.BlockSpec((HID, 1), lambda i: (0, 0)),
            pl.BlockSpec(memory_space=pltpu.SMEM),
            pl.BlockSpec((BLK, 1), lambda i: (i, 0)),
        ],
        out_specs=[
            pl.BlockSpec((BLK, HID), lambda i: (i, 0)),
            pl.BlockSpec((BLK, 1), lambda i: (i, 0)),
            pl.BlockSpec((G, 1), lambda i: (0, 0)),
        ],
        out_shape=[
            jax.ShapeDtypeStruct((N, HID), F32),
            jax.ShapeDtypeStruct((N, 1), F32),
            jax.ShapeDtypeStruct((G, 1), F32),
        ],
    )(ul, ur, dinv, x, sgw, sgb, a1w, a1b, a1a, gw1, gb1, ga, gw2, gb2,
      batch2d)


def _tc3_body(x_ref, x1_ref, gate_ref, gmax_ref, batch_ref, pw_ref, pb_ref,
              out_ref, racc, sacc):
    i = pl.program_id(0)

    @pl.when(i == 0)
    def _():
        racc[...] = jnp.zeros((G, 2 * HID), F32)
        sacc[...] = jnp.zeros((G, 1), F32)

    onehot = (batch_ref[...] ==
              lax.broadcasted_iota(jnp.int32, (BLK, G), 1)).astype(F32)
    gm = jnp.dot(onehot, gmax_ref[...], preferred_element_type=F32)
    e = jnp.exp(gate_ref[...] - gm)
    w = onehot * e
    feats = jnp.concatenate([x_ref[...], x1_ref[...]], axis=1)
    racc[...] += lax.dot_general(w, feats, (((0,), (0,)), ((), ())),
                                 preferred_element_type=F32)
    sacc[...] += jnp.sum(w, axis=0)[:, None]

    @pl.when(i == pl.num_programs(0) - 1)
    def _():
        reads = racc[...] / (sacc[...] + 1e-16)
        out_ref[...] = jnp.dot(reads, pw_ref[...],
                               preferred_element_type=F32) + pb_ref[...]


def _tc3(x, x1, gate, gmax, batch2d, pw, pb):
    return pl.pallas_call(
        _tc3_body,
        grid=(N // BLK,),
        in_specs=[
            pl.BlockSpec((BLK, HID), lambda i: (i, 0)),
            pl.BlockSpec((BLK, HID), lambda i: (i, 0)),
            pl.BlockSpec((BLK, 1), lambda i: (i, 0)),
            pl.BlockSpec((G, 1), lambda i: (0, 0)),
            pl.BlockSpec((BLK, 1), lambda i: (i, 0)),
            pl.BlockSpec((2 * HID, OUTC), lambda i: (0, 0)),
            pl.BlockSpec((1, OUTC), lambda i: (0, 0)),
        ],
        out_specs=pl.BlockSpec((G, OUTC), lambda i: (0, 0)),
        out_shape=jax.ShapeDtypeStruct((G, OUTC), F32),
        scratch_shapes=[
            pltpu.VMEM((G, 2 * HID), F32),
            pltpu.VMEM((G, 1), F32),
        ],
    )(x, x1, gate, gmax, batch2d, pw, pb)


OUTC = 128


def kernel(x, edge_index, batch, lin1_W, lin1_b, lin1_a, sg_W, sg_b,
           act1_W, act1_b, act1_a, gate_W1, gate_b1, gate_a, gate_W2,
           gate_b2, proj_W, proj_b):
    src = edge_index[0].reshape(NTILE, ESEG)
    dst = edge_index[1].reshape(NTILE, ESEG)
    padv = N + (jnp.arange(ESEGP - ESEG, dtype=jnp.int32) % (NPAD - N))
    pad2 = jnp.broadcast_to(padv, (NTILE, ESEGP - ESEG))
    srcp = jnp.concatenate([src, pad2], axis=1).reshape(NTILE, NCHUNK, CHUNK)
    dstp = jnp.concatenate([dst, pad2], axis=1).reshape(NTILE, NCHUNK, CHUNK)

    deg_k, prop_k = _sc_kernels()
    dega, degb = deg_k(dstp)

    g0l, g0r, invd, dinv = _tc1(
        x, lin1_W, lin1_b.reshape(1, HID), lin1_a.reshape(1, 1),
        dega.reshape(NPAD, 1), degb.reshape(NPAD, 1))

    ul, ur, _, _, _, _ = prop_k(g0l, g0r, srcp, dstp, invd.reshape(NPAD))

    batch2d = batch.reshape(N, 1)
    x1, gate, gmax = _tc2(
        ul[:N], ur[:N], dinv[:N], x, sg_W, sg_b.reshape(1, 2 * HID),
        act1_W, act1_b.reshape(1, HID), act1_a.reshape(1, 1),
        gate_W1, gate_b1.reshape(1, HID), gate_a.reshape(1, HID),
        gate_W2, gate_b2.reshape(1, 1), batch2d)

    return _tc3(x, x1, gate, gmax, batch2d, proj_W, proj_b.reshape(1, OUTC))


# speedup vs baseline: 18.5632x; 1.2606x over previous
import functools

import jax
import jax.numpy as jnp
from jax import lax
from jax.experimental import pallas as pl
from jax.experimental.pallas import tpu as pltpu
from jax.experimental.pallas import tpu_sc as plsc

N = 10000
NPAD = 10240
HID = 128
HALF = 64
G = 8
NTILE = 16
RPT = NPAD // NTILE
E = 320000
ESEG = E // NTILE
CHUNK = 128
NCHUNK = 157
ESEGP = NCHUNK * CHUNK
BLK = 1000
F32 = jnp.float32

_MESH = dict(core_axis_name="c", subcore_axis_name="s", num_cores=2,
             num_subcores=NTILE)


def _deg_body(dst_hbm, dega_hbm, degb_hbm, dst_v, ones_v, zline_v, deg_sh,
              sem):
    c = lax.axis_index("c")
    s = lax.axis_index("s")
    pltpu.sync_copy(dst_hbm.at[s], dst_v)
    for j in range(CHUNK // 16):
        ones_v[pl.ds(j * 16, 16)] = jnp.full((16,), 1.0, F32)
    for j in range(RPT // 16):
        zline_v[pl.ds(j * 16, 16)] = jnp.zeros((16,), F32)
    pltpu.sync_copy(zline_v, deg_sh.at[pl.ds(s * RPT, RPT)])
    plsc.subcore_barrier()

    nk = 79 - c

    def body(i, _):
        k = 2 * i + c
        pltpu.async_copy(ones_v, deg_sh.at[dst_v.at[k]], sem, add=True).wait()
        return 0

    lax.fori_loop(0, nk, body, 0)
    plsc.subcore_barrier()

    @pl.when(c == 0)
    def _():
        pltpu.sync_copy(deg_sh.at[pl.ds(s * RPT, RPT)],
                        dega_hbm.at[pl.ds(s * RPT, RPT)])

    @pl.when(c == 1)
    def _():
        pltpu.sync_copy(deg_sh.at[pl.ds(s * RPT, RPT)],
                        degb_hbm.at[pl.ds(s * RPT, RPT)])


SUB = 160
NSUB = RPT // SUB


def _prop_body(g0l, g0r, srci, dsti, invd, ul, ur, t1l, t1r, t2l, t2r,
               esrc_v, edst_v, invd_v, slab_v, gbufa_v, gbufb_v,
               gsa, gsb, ssa, ssb, acc_sh):
    c = lax.axis_index("c")
    s = lax.axis_index("s")
    row0 = s * RPT

    pltpu.sync_copy(srci.at[s], esrc_v)
    pltpu.sync_copy(dsti.at[s], edst_v)
    pltpu.sync_copy(invd.at[pl.ds(row0, RPT)], invd_v)

    for cval, g0 in ((0, g0l), (1, g0r)):
        @pl.when(c == cval)
        def _(g0=g0):
            for sub in range(NSUB):
                r0 = row0 + sub * SUB
                pltpu.sync_copy(g0.at[pl.ds(r0, SUB)], slab_v)
                pltpu.sync_copy(slab_v, acc_sh.at[pl.ds(r0, SUB)])
    plsc.subcore_barrier()

    srcs = ((g0l, t1l, t2l), (g0r, t1r, t2r))

    for r in range(3):
        for cval in (0, 1):
            gsrc = srcs[cval][r]

            @pl.when(c == cval)
            def _(gsrc=gsrc):
                def g_start(buf, sem_, k):
                    pltpu.async_copy(gsrc.at[esrc_v.at[k]], buf, sem_)

                def g_wait(buf, sem_, k):
                    pltpu.make_async_copy(gsrc.at[esrc_v.at[k]], buf,
                                          sem_).wait()

                def s_start(buf, sem_, k):
                    pltpu.async_copy(buf, acc_sh.at[edst_v.at[k]], sem_,
                                     add=True)

                def s_wait(buf, sem_, k):
                    pltpu.make_async_copy(buf, acc_sh.at[edst_v.at[k]],
                                          sem_).wait()

                g_start(gbufa_v, gsa, 0)

                def ebody(i, _):
                    k0 = 2 * i
                    k1 = k0 + 1
                    g_wait(gbufa_v, gsa, k0)

                    @pl.when(i > 0)
                    def _():
                        s_wait(gbufb_v, ssb, k0 - 1)

                    g_start(gbufb_v, gsb, k1)
                    s_start(gbufa_v, ssa, k0)
                    g_wait(gbufb_v, gsb, k1)
                    s_wait(gbufa_v, ssa, k0)
                    s_start(gbufb_v, ssb, k1)

                    @pl.when(k0 + 2 < NCHUNK)
                    def _():
                        g_start(gbufa_v, gsa, k0 + 2)

                    return 0

                lax.fori_loop(0, NCHUNK // 2, ebody, 0)
                g_wait(gbufa_v, gsa, NCHUNK - 1)
                s_wait(gbufb_v, ssb, NCHUNK - 2)
                s_start(gbufa_v, ssa, NCHUNK - 1)
                s_wait(gbufa_v, ssa, NCHUNK - 1)
        plsc.subcore_barrier()

        if r < 2:
            for sub in range(NSUB):
                r0 = row0 + sub * SUB
                pltpu.sync_copy(acc_sh.at[pl.ds(r0, SUB)], slab_v)

                def rbody(gi, _, sub=sub):
                    dv = invd_v[pl.ds(sub * SUB + gi * 16, 16)]
                    for rr in range(16):
                        d = dv[rr]
                        i = gi * 16 + rr
                        for j in range(HALF // 16):
                            sl = pl.ds(j * 16, 16)
                            slab_v[i, sl] = slab_v[i, sl] * d
                    return 0

                lax.fori_loop(0, SUB // 16, rbody, 0)
                pltpu.sync_copy(slab_v, acc_sh.at[pl.ds(r0, SUB)])
                for cval in (0, 1):
                    tnext = srcs[cval][r + 1]

                    @pl.when(c == cval)
                    def _(tnext=tnext):
                        pltpu.sync_copy(slab_v, tnext.at[pl.ds(r0, SUB)])
            plsc.subcore_barrier()
        else:
            for cval, uout in ((0, ul), (1, ur)):
                @pl.when(c == cval)
                def _(uout=uout):
                    pltpu.sync_copy(acc_sh.at[pl.ds(row0, RPT)],
                                    uout.at[pl.ds(row0, RPT)])


@functools.cache
def _sc_kernels():
    mesh = plsc.VectorSubcoreMesh(**_MESH)
    deg_k = pl.kernel(
        _deg_body,
        out_type=[jax.ShapeDtypeStruct((NPAD,), F32),
                  jax.ShapeDtypeStruct((NPAD,), F32)],
        mesh=mesh,
        scratch_types=[
            pltpu.VMEM((NCHUNK, CHUNK), jnp.int32),
            pltpu.VMEM((CHUNK,), F32),
            pltpu.VMEM((RPT,), F32),
            pltpu.VMEM_SHARED((NPAD,), F32),
            pltpu.SemaphoreType.DMA,
        ],
    )
    prop_k = pl.kernel(
        _prop_body,
        out_type=[jax.ShapeDtypeStruct((NPAD, HALF), F32)] * 6,
        mesh=mesh,
        scratch_types=[
            pltpu.VMEM((NCHUNK, CHUNK), jnp.int32),
            pltpu.VMEM((NCHUNK, CHUNK), jnp.int32),
            pltpu.VMEM((RPT,), F32),
            pltpu.VMEM((SUB, HALF), F32),
            pltpu.VMEM((CHUNK, HALF), F32),
            pltpu.VMEM((CHUNK, HALF), F32),
            pltpu.SemaphoreType.DMA,
            pltpu.SemaphoreType.DMA,
            pltpu.SemaphoreType.DMA,
            pltpu.SemaphoreType.DMA,
            pltpu.VMEM_SHARED((NPAD, HALF), F32),
        ],
        compiler_params=pltpu.CompilerParams(use_tc_tiling_on_sc=False),
    )
    return deg_k, prop_k


def _tc1_body(x_ref, w_ref, b_ref, a_ref, dega_ref, degb_ref,
              g0l_ref, g0r_ref, invd_ref, dinv_ref):
    h = jnp.dot(x_ref[...], w_ref[...], preferred_element_type=F32)
    h = h + b_ref[...]
    a = a_ref[0, 0]
    h = jnp.where(h >= 0, h, a * h)
    deg = dega_ref[...] + degb_ref[...] + 1.0
    dinv = lax.rsqrt(deg)
    g0 = h * dinv
    g0l_ref[...] = g0[:, :HALF]
    g0r_ref[...] = g0[:, HALF:]
    invd_ref[...] = 1.0 / deg
    dinv_ref[...] = dinv


def _tc1(x, w, b, a, dega, degb):
    return pl.pallas_call(
        _tc1_body,
        grid=(N // BLK,),
        in_specs=[
            pl.BlockSpec((BLK, HID), lambda i: (i, 0)),
            pl.BlockSpec((HID, HID), lambda i: (0, 0)),
            pl.BlockSpec((1, HID), lambda i: (0, 0)),
            pl.BlockSpec(memory_space=pltpu.SMEM),
            pl.BlockSpec((BLK, 1), lambda i: (i, 0)),
            pl.BlockSpec((BLK, 1), lambda i: (i, 0)),
        ],
        out_specs=[
            pl.BlockSpec((BLK, HALF), lambda i: (i, 0)),
            pl.BlockSpec((BLK, HALF), lambda i: (i, 0)),
            pl.BlockSpec((BLK, 1), lambda i: (i, 0)),
            pl.BlockSpec((BLK, 1), lambda i: (i, 0)),
        ],
        out_shape=[
            jax.ShapeDtypeStruct((NPAD, HALF), F32),
            jax.ShapeDtypeStruct((NPAD, HALF), F32),
            jax.ShapeDtypeStruct((NPAD, 1), F32),
            jax.ShapeDtypeStruct((NPAD, 1), F32),
        ],
    )(x, w, b, a, dega, degb)


def _tc2_body(ul_ref, ur_ref, dinv_ref, x_ref, sgw_ref, sgb_ref,
              a1w_ref, a1b_ref, a1a_ref, gw1_ref, gb1_ref, ga_ref,
              gw2_ref, gb2_ref, batch_ref, x1_ref, gate_ref, gmax_ref):
    i = pl.program_id(0)
    u = jnp.concatenate([ul_ref[...], ur_ref[...]], axis=1)
    h3 = u * dinv_ref[...]
    h = jnp.dot(h3, sgw_ref[...], preferred_element_type=F32) + sgb_ref[...]
    x1 = jnp.dot(h, a1w_ref[...], preferred_element_type=F32) + a1b_ref[...]
    a1 = a1a_ref[0, 0]
    x1 = jnp.where(x1 >= 0, x1, a1 * x1)
    x1_ref[...] = x1
    feats = jnp.concatenate([x_ref[...], x1], axis=1)
    g1 = jnp.dot(feats, gw1_ref[...], preferred_element_type=F32)
    g1 = g1 + gb1_ref[...]
    g1 = jnp.where(g1 >= 0, g1, ga_ref[...] * g1)
    gate = jnp.dot(g1, gw2_ref[...], preferred_element_type=F32)
    gate = gate + gb2_ref[0, 0]
    gate_ref[...] = gate

    onehot = batch_ref[...] == lax.broadcasted_iota(jnp.int32, (BLK, G), 1)
    gm = jnp.max(jnp.where(onehot, gate, jnp.float32(-3e38)), axis=0)[:, None]

    @pl.when(i == 0)
    def _():
        gmax_ref[...] = jnp.full((G, 1), -3e38, F32)

    gmax_ref[...] = jnp.maximum(gmax_ref[...], gm)


def _tc2(ul, ur, dinv, x, sgw, sgb, a1w, a1b, a1a, gw1, gb1, ga, gw2, gb2,
         batch2d):
    return pl.pallas_call(
        _tc2_body,
        grid=(N // BLK,),
        in_specs=[
            pl.BlockSpec((BLK, HALF), lambda i: (i, 0)),
            pl.BlockSpec((BLK, HALF), lambda i: (i, 0)),
            pl.BlockSpec((BLK, 1), lambda i: (i, 0)),
            pl.BlockSpec((BLK, HID), lambda i: (i, 0)),
            pl.BlockSpec((HID, 2 * HID), lambda i: (0, 0)),
            pl.BlockSpec((1, 2 * HID), lambda i: (0, 0)),
            pl.BlockSpec((2 * HID, HID), lambda i: (0, 0)),
            pl.BlockSpec((1, HID), lambda i: (0, 0)),
            pl.BlockSpec(memory_space=pltpu.SMEM),
            pl.BlockSpec((2 * HID, HID), lambda i: (0, 0)),
            pl.BlockSpec((1, HID), lambda i: (0, 0)),
            pl.BlockSpec((1, HID), lambda i: (0, 0)),
            pl.BlockSpec((HID, 1), lambda i: (0, 0)),
            pl.BlockSpec(memory_space=pltpu.SMEM),
            pl.BlockSpec((BLK, 1), lambda i: (i, 0)),
        ],
        out_specs=[
            pl.BlockSpec((BLK, HID), lambda i: (i, 0)),
            pl.BlockSpec((BLK, 1), lambda i: (i, 0)),
            pl.BlockSpec((G, 1), lambda i: (0, 0)),
        ],
        out_shape=[
            jax.ShapeDtypeStruct((N, HID), F32),
            jax.ShapeDtypeStruct((N, 1), F32),
            jax.ShapeDtypeStruct((G, 1), F32),
        ],
    )(ul, ur, dinv, x, sgw, sgb, a1w, a1b, a1a, gw1, gb1, ga, gw2, gb2,
      batch2d)


def _tc3_body(x_ref, x1_ref, gate_ref, gmax_ref, batch_ref, pw_ref, pb_ref,
              out_ref, racc, sacc):
    i = pl.program_id(0)

    @pl.when(i == 0)
    def _():
        racc[...] = jnp.zeros((G, 2 * HID), F32)
        sacc[...] = jnp.zeros((G, 1), F32)

    onehot = (batch_ref[...] ==
              lax.broadcasted_iota(jnp.int32, (BLK, G), 1)).astype(F32)
    gm = jnp.dot(onehot, gmax_ref[...], preferred_element_type=F32)
    e = jnp.exp(gate_ref[...] - gm)
    w = onehot * e
    feats = jnp.concatenate([x_ref[...], x1_ref[...]], axis=1)
    racc[...] += lax.dot_general(w, feats, (((0,), (0,)), ((), ())),
                                 preferred_element_type=F32)
    sacc[...] += jnp.sum(w, axis=0)[:, None]

    @pl.when(i == pl.num_programs(0) - 1)
    def _():
        reads = racc[...] / (sacc[...] + 1e-16)
        out_ref[...] = jnp.dot(reads, pw_ref[...],
                               preferred_element_type=F32) + pb_ref[...]


def _tc3(x, x1, gate, gmax, batch2d, pw, pb):
    return pl.pallas_call(
        _tc3_body,
        grid=(N // BLK,),
        in_specs=[
            pl.BlockSpec((BLK, HID), lambda i: (i, 0)),
            pl.BlockSpec((BLK, HID), lambda i: (i, 0)),
            pl.BlockSpec((BLK, 1), lambda i: (i, 0)),
            pl.BlockSpec((G, 1), lambda i: (0, 0)),
            pl.BlockSpec((BLK, 1), lambda i: (i, 0)),
            pl.BlockSpec((2 * HID, OUTC), lambda i: (0, 0)),
            pl.BlockSpec((1, OUTC), lambda i: (0, 0)),
        ],
        out_specs=pl.BlockSpec((G, OUTC), lambda i: (0, 0)),
        out_shape=jax.ShapeDtypeStruct((G, OUTC), F32),
        scratch_shapes=[
            pltpu.VMEM((G, 2 * HID), F32),
            pltpu.VMEM((G, 1), F32),
        ],
    )(x, x1, gate, gmax, batch2d, pw, pb)


OUTC = 128


def kernel(x, edge_index, batch, lin1_W, lin1_b, lin1_a, sg_W, sg_b,
           act1_W, act1_b, act1_a, gate_W1, gate_b1, gate_a, gate_W2,
           gate_b2, proj_W, proj_b):
    src = edge_index[0].reshape(NTILE, ESEG)
    dst = edge_index[1].reshape(NTILE, ESEG)
    padv = N + (jnp.arange(ESEGP - ESEG, dtype=jnp.int32) % (NPAD - N))
    pad2 = jnp.broadcast_to(padv, (NTILE, ESEGP - ESEG))
    srcp = jnp.concatenate([src, pad2], axis=1).reshape(NTILE, NCHUNK, CHUNK)
    dstp = jnp.concatenate([dst, pad2], axis=1).reshape(NTILE, NCHUNK, CHUNK)

    deg_k, prop_k = _sc_kernels()
    dega, degb = deg_k(dstp)

    g0l, g0r, invd, dinv = _tc1(
        x, lin1_W, lin1_b.reshape(1, HID), lin1_a.reshape(1, 1),
        dega.reshape(NPAD, 1), degb.reshape(NPAD, 1))

    ul, ur, _, _, _, _ = prop_k(g0l, g0r, srcp, dstp, invd.reshape(NPAD))

    batch2d = batch.reshape(N, 1)
    x1, gate, gmax = _tc2(
        ul[:N], ur[:N], dinv[:N], x, sg_W, sg_b.reshape(1, 2 * HID),
        act1_W, act1_b.reshape(1, HID), act1_a.reshape(1, 1),
        gate_W1, gate_b1.reshape(1, HID), gate_a.reshape(1, HID),
        gate_W2, gate_b2.reshape(1, 1), batch2d)

    return _tc3(x, x1, gate, gmax, batch2d, proj_W, proj_b.reshape(1, OUTC))


# speedup vs baseline: 27.2642x; 1.4687x over previous
import functools

import jax
import jax.numpy as jnp
from jax import lax
from jax.experimental import pallas as pl
from jax.experimental.pallas import tpu as pltpu
from jax.experimental.pallas import tpu_sc as plsc

N = 10000
NPAD = 10240
HID = 128
HALF = 64
G = 8
NTILE = 16
RPT = NPAD // NTILE
E = 320000
ESEG = E // NTILE
CHUNK = 128
NCHUNK = 157
ESEGP = NCHUNK * CHUNK
BLK = 1000
F32 = jnp.float32

_MESH = dict(core_axis_name="c", subcore_axis_name="s", num_cores=2,
             num_subcores=NTILE)


def _deg_body(dst_hbm, dega_hbm, degb_hbm, dst_v, ones_v, zline_v, deg_sh,
              sem):
    c = lax.axis_index("c")
    s = lax.axis_index("s")
    pltpu.sync_copy(dst_hbm.at[s], dst_v)
    for j in range(CHUNK // 16):
        ones_v[pl.ds(j * 16, 16)] = jnp.full((16,), 1.0, F32)
    for j in range(RPT // 16):
        zline_v[pl.ds(j * 16, 16)] = jnp.zeros((16,), F32)
    pltpu.sync_copy(zline_v, deg_sh.at[pl.ds(s * RPT, RPT)])
    plsc.subcore_barrier()

    nk = 79 - c

    def body(i, _):
        k = 2 * i + c
        pltpu.async_copy(ones_v, deg_sh.at[dst_v.at[k]], sem, add=True).wait()
        return 0

    lax.fori_loop(0, nk, body, 0)
    plsc.subcore_barrier()

    @pl.when(c == 0)
    def _():
        pltpu.sync_copy(deg_sh.at[pl.ds(s * RPT, RPT)],
                        dega_hbm.at[pl.ds(s * RPT, RPT)])

    @pl.when(c == 1)
    def _():
        pltpu.sync_copy(deg_sh.at[pl.ds(s * RPT, RPT)],
                        degb_hbm.at[pl.ds(s * RPT, RPT)])


SUB = 160
NSUB = RPT // SUB
NBUF = 4


def _prop_body(g0l, g0r, srci, dsti, invd, ul, ur, t1l, t1r, t2l, t2r,
               esrc_v, dbuf_v, invd_v, slab_v, gbuf_v,
               gsem, ssem, dsem, acc_sh):
    c = lax.axis_index("c")
    s = lax.axis_index("s")
    row0 = s * RPT

    pltpu.sync_copy(srci.at[s], esrc_v)
    pltpu.sync_copy(invd.at[pl.ds(row0, RPT)], invd_v)

    for cval, g0 in ((0, g0l), (1, g0r)):
        @pl.when(c == cval)
        def _(g0=g0):
            for sub in range(NSUB):
                r0 = row0 + sub * SUB
                pltpu.sync_copy(g0.at[pl.ds(r0, SUB)], slab_v)
                pltpu.sync_copy(slab_v, acc_sh.at[pl.ds(r0, SUB)])
    plsc.subcore_barrier()

    srcs = ((g0l, t1l, t2l), (g0r, t1r, t2r))

    for r in range(3):
        for cval in (0, 1):
            gsrc = srcs[cval][r]

            @pl.when(c == cval)
            def _(gsrc=gsrc):
                def d_start(b, k):
                    pltpu.async_copy(dsti.at[s, k], dbuf_v.at[b],
                                     dsem.at[b])

                def d_wait(b, k):
                    pltpu.make_async_copy(dsti.at[s, k], dbuf_v.at[b],
                                          dsem.at[b]).wait()

                def g_start(b, k):
                    pltpu.async_copy(gsrc.at[esrc_v.at[k]], gbuf_v.at[b],
                                     gsem.at[b])

                def g_wait(b, k):
                    pltpu.make_async_copy(gsrc.at[esrc_v.at[k]],
                                          gbuf_v.at[b], gsem.at[b]).wait()

                def s_start(b, k):
                    pltpu.async_copy(gbuf_v.at[b], acc_sh.at[dbuf_v.at[b]],
                                     ssem.at[b], add=True)

                def s_wait(b, k):
                    pltpu.make_async_copy(gbuf_v.at[b],
                                          acc_sh.at[dbuf_v.at[b]],
                                          ssem.at[b]).wait()

                for p in range(2):
                    d_start(p, p)
                    g_start(p, p)

                def ebody(k, _):
                    b = lax.rem(k, NBUF)
                    b2 = lax.rem(k + 2, NBUF)

                    @pl.when(k >= 2)
                    def _():
                        s_wait(b2, k - 2)

                    @pl.when(k + 2 < NCHUNK)
                    def _():
                        d_start(b2, k + 2)
                        g_start(b2, k + 2)

                    g_wait(b, k)
                    d_wait(b, k)
                    s_start(b, k)
                    return 0

                lax.fori_loop(0, NCHUNK, ebody, 0)
                s_wait((NCHUNK - 2) % NBUF, NCHUNK - 2)
                s_wait((NCHUNK - 1) % NBUF, NCHUNK - 1)
        plsc.subcore_barrier()

        if r < 2:
            for sub in range(NSUB):
                r0 = row0 + sub * SUB
                pltpu.sync_copy(acc_sh.at[pl.ds(r0, SUB)], slab_v)

                def rbody(gi, _, sub=sub):
                    dv = invd_v[pl.ds(sub * SUB + gi * 16, 16)]
                    for rr in range(16):
                        d = dv[rr]
                        i = gi * 16 + rr
                        for j in range(HALF // 16):
                            sl = pl.ds(j * 16, 16)
                            slab_v[i, sl] = slab_v[i, sl] * d
                    return 0

                lax.fori_loop(0, SUB // 16, rbody, 0)
                pltpu.sync_copy(slab_v, acc_sh.at[pl.ds(r0, SUB)])
                for cval in (0, 1):
                    tnext = srcs[cval][r + 1]

                    @pl.when(c == cval)
                    def _(tnext=tnext):
                        pltpu.sync_copy(slab_v, tnext.at[pl.ds(r0, SUB)])
            plsc.subcore_barrier()
        else:
            for cval, uout in ((0, ul), (1, ur)):
                @pl.when(c == cval)
                def _(uout=uout):
                    pltpu.sync_copy(acc_sh.at[pl.ds(row0, RPT)],
                                    uout.at[pl.ds(row0, RPT)])


@functools.cache
def _sc_kernels():
    mesh = plsc.VectorSubcoreMesh(**_MESH)
    deg_k = pl.kernel(
        _deg_body,
        out_type=[jax.ShapeDtypeStruct((NPAD,), F32),
                  jax.ShapeDtypeStruct((NPAD,), F32)],
        mesh=mesh,
        scratch_types=[
            pltpu.VMEM((NCHUNK, CHUNK), jnp.int32),
            pltpu.VMEM((CHUNK,), F32),
            pltpu.VMEM((RPT,), F32),
            pltpu.VMEM_SHARED((NPAD,), F32),
            pltpu.SemaphoreType.DMA,
        ],
    )
    prop_k = pl.kernel(
        _prop_body,
        out_type=[jax.ShapeDtypeStruct((NPAD, HALF), F32)] * 6,
        mesh=mesh,
        scratch_types=[
            pltpu.VMEM((NCHUNK, CHUNK), jnp.int32),
            pltpu.VMEM((NBUF, CHUNK), jnp.int32),
            pltpu.VMEM((RPT,), F32),
            pltpu.VMEM((SUB, HALF), F32),
            pltpu.VMEM((NBUF, CHUNK, HALF), F32),
            pltpu.SemaphoreType.DMA((NBUF,)),
            pltpu.SemaphoreType.DMA((NBUF,)),
            pltpu.SemaphoreType.DMA((NBUF,)),
            pltpu.VMEM_SHARED((NPAD, HALF), F32),
        ],
        compiler_params=pltpu.CompilerParams(use_tc_tiling_on_sc=False),
    )
    return deg_k, prop_k


def _tc1_body(x_ref, w_ref, b_ref, a_ref, dega_ref, degb_ref,
              g0l_ref, g0r_ref, invd_ref, dinv_ref):
    h = jnp.dot(x_ref[...], w_ref[...], preferred_element_type=F32)
    h = h + b_ref[...]
    a = a_ref[0, 0]
    h = jnp.where(h >= 0, h, a * h)
    deg = dega_ref[...] + degb_ref[...] + 1.0
    dinv = lax.rsqrt(deg)
    g0 = h * dinv
    g0l_ref[...] = g0[:, :HALF]
    g0r_ref[...] = g0[:, HALF:]
    invd_ref[...] = 1.0 / deg
    dinv_ref[...] = dinv


def _tc1(x, w, b, a, dega, degb):
    return pl.pallas_call(
        _tc1_body,
        grid=(N // BLK,),
        in_specs=[
            pl.BlockSpec((BLK, HID), lambda i: (i, 0)),
            pl.BlockSpec((HID, HID), lambda i: (0, 0)),
            pl.BlockSpec((1, HID), lambda i: (0, 0)),
            pl.BlockSpec(memory_space=pltpu.SMEM),
            pl.BlockSpec((BLK, 1), lambda i: (i, 0)),
            pl.BlockSpec((BLK, 1), lambda i: (i, 0)),
        ],
        out_specs=[
            pl.BlockSpec((BLK, HALF), lambda i: (i, 0)),
            pl.BlockSpec((BLK, HALF), lambda i: (i, 0)),
            pl.BlockSpec((BLK, 1), lambda i: (i, 0)),
            pl.BlockSpec((BLK, 1), lambda i: (i, 0)),
        ],
        out_shape=[
            jax.ShapeDtypeStruct((NPAD, HALF), F32),
            jax.ShapeDtypeStruct((NPAD, HALF), F32),
            jax.ShapeDtypeStruct((NPAD, 1), F32),
            jax.ShapeDtypeStruct((NPAD, 1), F32),
        ],
    )(x, w, b, a, dega, degb)


def _tc2_body(ul_ref, ur_ref, dinv_ref, x_ref, sgw_ref, sgb_ref,
              a1w_ref, a1b_ref, a1a_ref, gw1_ref, gb1_ref, ga_ref,
              gw2_ref, gb2_ref, batch_ref, x1_ref, gate_ref, gmax_ref):
    i = pl.program_id(0)
    u = jnp.concatenate([ul_ref[...], ur_ref[...]], axis=1)
    h3 = u * dinv_ref[...]
    h = jnp.dot(h3, sgw_ref[...], preferred_element_type=F32) + sgb_ref[...]
    x1 = jnp.dot(h, a1w_ref[...], preferred_element_type=F32) + a1b_ref[...]
    a1 = a1a_ref[0, 0]
    x1 = jnp.where(x1 >= 0, x1, a1 * x1)
    x1_ref[...] = x1
    feats = jnp.concatenate([x_ref[...], x1], axis=1)
    g1 = jnp.dot(feats, gw1_ref[...], preferred_element_type=F32)
    g1 = g1 + gb1_ref[...]
    g1 = jnp.where(g1 >= 0, g1, ga_ref[...] * g1)
    gate = jnp.dot(g1, gw2_ref[...], preferred_element_type=F32)
    gate = gate + gb2_ref[0, 0]
    gate_ref[...] = gate

    onehot = batch_ref[...] == lax.broadcasted_iota(jnp.int32, (BLK, G), 1)
    gm = jnp.max(jnp.where(onehot, gate, jnp.float32(-3e38)), axis=0)[:, None]

    @pl.when(i == 0)
    def _():
        gmax_ref[...] = jnp.full((G, 1), -3e38, F32)

    gmax_ref[...] = jnp.maximum(gmax_ref[...], gm)


def _tc2(ul, ur, dinv, x, sgw, sgb, a1w, a1b, a1a, gw1, gb1, ga, gw2, gb2,
         batch2d):
    return pl.pallas_call(
        _tc2_body,
        grid=(N // BLK,),
        in_specs=[
            pl.BlockSpec((BLK, HALF), lambda i: (i, 0)),
            pl.BlockSpec((BLK, HALF), lambda i: (i, 0)),
            pl.BlockSpec((BLK, 1), lambda i: (i, 0)),
            pl.BlockSpec((BLK, HID), lambda i: (i, 0)),
            pl.BlockSpec((HID, 2 * HID), lambda i: (0, 0)),
            pl.BlockSpec((1, 2 * HID), lambda i: (0, 0)),
            pl.BlockSpec((2 * HID, HID), lambda i: (0, 0)),
            pl.BlockSpec((1, HID), lambda i: (0, 0)),
            pl.BlockSpec(memory_space=pltpu.SMEM),
            pl.BlockSpec((2 * HID, HID), lambda i: (0, 0)),
            pl.BlockSpec((1, HID), lambda i: (0, 0)),
            pl.BlockSpec((1, HID), lambda i: (0, 0)),
            pl.BlockSpec((HID, 1), lambda i: (0, 0)),
            pl.BlockSpec(memory_space=pltpu.SMEM),
            pl.BlockSpec((BLK, 1), lambda i: (i, 0)),
        ],
        out_specs=[
            pl.BlockSpec((BLK, HID), lambda i: (i, 0)),
            pl.BlockSpec((BLK, 1), lambda i: (i, 0)),
            pl.BlockSpec((G, 1), lambda i: (0, 0)),
        ],
        out_shape=[
            jax.ShapeDtypeStruct((N, HID), F32),
            jax.ShapeDtypeStruct((N, 1), F32),
            jax.ShapeDtypeStruct((G, 1), F32),
        ],
    )(ul, ur, dinv, x, sgw, sgb, a1w, a1b, a1a, gw1, gb1, ga, gw2, gb2,
      batch2d)


def _tc3_body(x_ref, x1_ref, gate_ref, gmax_ref, batch_ref, pw_ref, pb_ref,
              out_ref, racc, sacc):
    i = pl.program_id(0)

    @pl.when(i == 0)
    def _():
        racc[...] = jnp.zeros((G, 2 * HID), F32)
        sacc[...] = jnp.zeros((G, 1), F32)

    onehot = (batch_ref[...] ==
              lax.broadcasted_iota(jnp.int32, (BLK, G), 1)).astype(F32)
    gm = jnp.dot(onehot, gmax_ref[...], preferred_element_type=F32)
    e = jnp.exp(gate_ref[...] - gm)
    w = onehot * e
    feats = jnp.concatenate([x_ref[...], x1_ref[...]], axis=1)
    racc[...] += lax.dot_general(w, feats, (((0,), (0,)), ((), ())),
                                 preferred_element_type=F32)
    sacc[...] += jnp.sum(w, axis=0)[:, None]

    @pl.when(i == pl.num_programs(0) - 1)
    def _():
        reads = racc[...] / (sacc[...] + 1e-16)
        out_ref[...] = jnp.dot(reads, pw_ref[...],
                               preferred_element_type=F32) + pb_ref[...]


def _tc3(x, x1, gate, gmax, batch2d, pw, pb):
    return pl.pallas_call(
        _tc3_body,
        grid=(N // BLK,),
        in_specs=[
            pl.BlockSpec((BLK, HID), lambda i: (i, 0)),
            pl.BlockSpec((BLK, HID), lambda i: (i, 0)),
            pl.BlockSpec((BLK, 1), lambda i: (i, 0)),
            pl.BlockSpec((G, 1), lambda i: (0, 0)),
            pl.BlockSpec((BLK, 1), lambda i: (i, 0)),
            pl.BlockSpec((2 * HID, OUTC), lambda i: (0, 0)),
            pl.BlockSpec((1, OUTC), lambda i: (0, 0)),
        ],
        out_specs=pl.BlockSpec((G, OUTC), lambda i: (0, 0)),
        out_shape=jax.ShapeDtypeStruct((G, OUTC), F32),
        scratch_shapes=[
            pltpu.VMEM((G, 2 * HID), F32),
            pltpu.VMEM((G, 1), F32),
        ],
    )(x, x1, gate, gmax, batch2d, pw, pb)


OUTC = 128


def kernel(x, edge_index, batch, lin1_W, lin1_b, lin1_a, sg_W, sg_b,
           act1_W, act1_b, act1_a, gate_W1, gate_b1, gate_a, gate_W2,
           gate_b2, proj_W, proj_b):
    src = edge_index[0].reshape(NTILE, ESEG)
    dst = edge_index[1].reshape(NTILE, ESEG)
    padv = N + (jnp.arange(ESEGP - ESEG, dtype=jnp.int32) % (NPAD - N))
    pad2 = jnp.broadcast_to(padv, (NTILE, ESEGP - ESEG))
    srcp = jnp.concatenate([src, pad2], axis=1).reshape(NTILE, NCHUNK, CHUNK)
    dstp = jnp.concatenate([dst, pad2], axis=1).reshape(NTILE, NCHUNK, CHUNK)

    deg_k, prop_k = _sc_kernels()
    dega, degb = deg_k(dstp)

    g0l, g0r, invd, dinv = _tc1(
        x, lin1_W, lin1_b.reshape(1, HID), lin1_a.reshape(1, 1),
        dega.reshape(NPAD, 1), degb.reshape(NPAD, 1))

    ul, ur, _, _, _, _ = prop_k(g0l, g0r, srcp, dstp, invd.reshape(NPAD))

    batch2d = batch.reshape(N, 1)
    x1, gate, gmax = _tc2(
        ul[:N], ur[:N], dinv[:N], x, sg_W, sg_b.reshape(1, 2 * HID),
        act1_W, act1_b.reshape(1, HID), act1_a.reshape(1, 1),
        gate_W1, gate_b1.reshape(1, HID), gate_a.reshape(1, HID),
        gate_W2, gate_b2.reshape(1, 1), batch2d)

    return _tc3(x, x1, gate, gmax, batch2d, proj_W, proj_b.reshape(1, OUTC))


# speedup vs baseline: 28.8237x; 1.0572x over previous
import functools

import jax
import jax.numpy as jnp
from jax import lax
from jax.experimental import pallas as pl
from jax.experimental.pallas import tpu as pltpu
from jax.experimental.pallas import tpu_sc as plsc

N = 10000
NPAD = 10240
HID = 128
HALF = 64
G = 8
NTILE = 16
RPT = NPAD // NTILE
E = 320000
ESEG = E // NTILE
CHUNK = 128
NCHUNK = 157
ESEGP = NCHUNK * CHUNK
BLK = 1000
F32 = jnp.float32

_MESH = dict(core_axis_name="c", subcore_axis_name="s", num_cores=2,
             num_subcores=NTILE)


def _deg_body(dst_hbm, dega_hbm, degb_hbm, dst_v, ones_v, zline_v, deg_sh,
              sem):
    c = lax.axis_index("c")
    s = lax.axis_index("s")
    pltpu.sync_copy(dst_hbm.at[s], dst_v)
    for j in range(CHUNK // 16):
        ones_v[pl.ds(j * 16, 16)] = jnp.full((16,), 1.0, F32)
    for j in range(RPT // 16):
        zline_v[pl.ds(j * 16, 16)] = jnp.zeros((16,), F32)
    pltpu.sync_copy(zline_v, deg_sh.at[pl.ds(s * RPT, RPT)])
    plsc.subcore_barrier()

    nk = 79 - c

    def body(i, _):
        k = 2 * i + c
        pltpu.async_copy(ones_v, deg_sh.at[dst_v.at[k]], sem, add=True).wait()
        return 0

    lax.fori_loop(0, nk, body, 0)
    plsc.subcore_barrier()

    @pl.when(c == 0)
    def _():
        pltpu.sync_copy(deg_sh.at[pl.ds(s * RPT, RPT)],
                        dega_hbm.at[pl.ds(s * RPT, RPT)])

    @pl.when(c == 1)
    def _():
        pltpu.sync_copy(deg_sh.at[pl.ds(s * RPT, RPT)],
                        degb_hbm.at[pl.ds(s * RPT, RPT)])


SUB = 80
NSUB = RPT // SUB
NBUF = 5
LEAD = 3


def _prop_body(g0l, g0r, srci, dsti, invd, ul, ur, t1l, t1r, t2l, t2r,
               esrc_v, dbuf_v, invd_v, slab_v, gbuf_v,
               gsem, ssem, dsem, acc_sh):
    c = lax.axis_index("c")
    s = lax.axis_index("s")
    row0 = s * RPT

    pltpu.sync_copy(srci.at[s], esrc_v)
    pltpu.sync_copy(invd.at[pl.ds(row0, RPT)], invd_v)

    for cval, g0 in ((0, g0l), (1, g0r)):
        @pl.when(c == cval)
        def _(g0=g0):
            for sub in range(NSUB):
                r0 = row0 + sub * SUB
                pltpu.sync_copy(g0.at[pl.ds(r0, SUB)], slab_v)
                pltpu.sync_copy(slab_v, acc_sh.at[pl.ds(r0, SUB)])
    plsc.subcore_barrier()

    srcs = ((g0l, t1l, t2l), (g0r, t1r, t2r))

    for r in range(3):
        for cval in (0, 1):
            gsrc = srcs[cval][r]

            @pl.when(c == cval)
            def _(gsrc=gsrc):
                def d_start(b, k):
                    pltpu.async_copy(dsti.at[s, k], dbuf_v.at[b],
                                     dsem.at[b])

                def d_wait(b, k):
                    pltpu.make_async_copy(dsti.at[s, k], dbuf_v.at[b],
                                          dsem.at[b]).wait()

                def g_start(b, k):
                    pltpu.async_copy(gsrc.at[esrc_v.at[k]], gbuf_v.at[b],
                                     gsem.at[b])

                def g_wait(b, k):
                    pltpu.make_async_copy(gsrc.at[esrc_v.at[k]],
                                          gbuf_v.at[b], gsem.at[b]).wait()

                def s_start(b, k):
                    pltpu.async_copy(gbuf_v.at[b], acc_sh.at[dbuf_v.at[b]],
                                     ssem.at[b], add=True)

                def s_wait(b, k):
                    pltpu.make_async_copy(gbuf_v.at[b],
                                          acc_sh.at[dbuf_v.at[b]],
                                          ssem.at[b]).wait()

                for p in range(LEAD):
                    d_start(p, p)
                    g_start(p, p)

                def ebody(k, _):
                    b = lax.rem(k, NBUF)
                    bl = lax.rem(k + LEAD, NBUF)

                    @pl.when(k >= NBUF - LEAD)
                    def _():
                        s_wait(bl, k - (NBUF - LEAD))

                    @pl.when(k + LEAD < NCHUNK)
                    def _():
                        d_start(bl, k + LEAD)
                        g_start(bl, k + LEAD)

                    g_wait(b, k)
                    d_wait(b, k)
                    s_start(b, k)
                    return 0

                lax.fori_loop(0, NCHUNK, ebody, 0)
                for t in range(NBUF - LEAD):
                    kk = NCHUNK - (NBUF - LEAD) + t
                    s_wait(kk % NBUF, kk)
        plsc.subcore_barrier()

        if r < 2:
            for sub in range(NSUB):
                r0 = row0 + sub * SUB
                pltpu.sync_copy(acc_sh.at[pl.ds(r0, SUB)], slab_v)

                def rbody(gi, _, sub=sub):
                    dv = invd_v[pl.ds(sub * SUB + gi * 16, 16)]
                    for rr in range(16):
                        d = dv[rr]
                        i = gi * 16 + rr
                        for j in range(HALF // 16):
                            sl = pl.ds(j * 16, 16)
                            slab_v[i, sl] = slab_v[i, sl] * d
                    return 0

                lax.fori_loop(0, SUB // 16, rbody, 0)
                pltpu.sync_copy(slab_v, acc_sh.at[pl.ds(r0, SUB)])
                for cval in (0, 1):
                    tnext = srcs[cval][r + 1]

                    @pl.when(c == cval)
                    def _(tnext=tnext):
                        pltpu.sync_copy(slab_v, tnext.at[pl.ds(r0, SUB)])
            plsc.subcore_barrier()
        else:
            for cval, uout in ((0, ul), (1, ur)):
                @pl.when(c == cval)
                def _(uout=uout):
                    pltpu.sync_copy(acc_sh.at[pl.ds(row0, RPT)],
                                    uout.at[pl.ds(row0, RPT)])


@functools.cache
def _sc_kernels():
    mesh = plsc.VectorSubcoreMesh(**_MESH)
    deg_k = pl.kernel(
        _deg_body,
        out_type=[jax.ShapeDtypeStruct((NPAD,), F32),
                  jax.ShapeDtypeStruct((NPAD,), F32)],
        mesh=mesh,
        scratch_types=[
            pltpu.VMEM((NCHUNK, CHUNK), jnp.int32),
            pltpu.VMEM((CHUNK,), F32),
            pltpu.VMEM((RPT,), F32),
            pltpu.VMEM_SHARED((NPAD,), F32),
            pltpu.SemaphoreType.DMA,
        ],
    )
    prop_k = pl.kernel(
        _prop_body,
        out_type=[jax.ShapeDtypeStruct((NPAD, HALF), F32)] * 6,
        mesh=mesh,
        scratch_types=[
            pltpu.VMEM((NCHUNK, CHUNK), jnp.int32),
            pltpu.VMEM((NBUF, CHUNK), jnp.int32),
            pltpu.VMEM((RPT,), F32),
            pltpu.VMEM((SUB, HALF), F32),
            pltpu.VMEM((NBUF, CHUNK, HALF), F32),
            pltpu.SemaphoreType.DMA((NBUF,)),
            pltpu.SemaphoreType.DMA((NBUF,)),
            pltpu.SemaphoreType.DMA((NBUF,)),
            pltpu.VMEM_SHARED((NPAD, HALF), F32),
        ],
        compiler_params=pltpu.CompilerParams(use_tc_tiling_on_sc=False),
    )
    return deg_k, prop_k


def _tc1_body(x_ref, w_ref, b_ref, a_ref, dega_ref, degb_ref,
              g0l_ref, g0r_ref, invd_ref, dinv_ref):
    h = jnp.dot(x_ref[...], w_ref[...], preferred_element_type=F32)
    h = h + b_ref[...]
    a = a_ref[0, 0]
    h = jnp.where(h >= 0, h, a * h)
    deg = dega_ref[...] + degb_ref[...] + 1.0
    dinv = lax.rsqrt(deg)
    g0 = h * dinv
    g0l_ref[...] = g0[:, :HALF]
    g0r_ref[...] = g0[:, HALF:]
    invd_ref[...] = 1.0 / deg
    dinv_ref[...] = dinv


def _tc1(x, w, b, a, dega, degb):
    return pl.pallas_call(
        _tc1_body,
        grid=(N // BLK,),
        in_specs=[
            pl.BlockSpec((BLK, HID), lambda i: (i, 0)),
            pl.BlockSpec((HID, HID), lambda i: (0, 0)),
            pl.BlockSpec((1, HID), lambda i: (0, 0)),
            pl.BlockSpec(memory_space=pltpu.SMEM),
            pl.BlockSpec((BLK, 1), lambda i: (i, 0)),
            pl.BlockSpec((BLK, 1), lambda i: (i, 0)),
        ],
        out_specs=[
            pl.BlockSpec((BLK, HALF), lambda i: (i, 0)),
            pl.BlockSpec((BLK, HALF), lambda i: (i, 0)),
            pl.BlockSpec((BLK, 1), lambda i: (i, 0)),
            pl.BlockSpec((BLK, 1), lambda i: (i, 0)),
        ],
        out_shape=[
            jax.ShapeDtypeStruct((NPAD, HALF), F32),
            jax.ShapeDtypeStruct((NPAD, HALF), F32),
            jax.ShapeDtypeStruct((NPAD, 1), F32),
            jax.ShapeDtypeStruct((NPAD, 1), F32),
        ],
    )(x, w, b, a, dega, degb)


def _tc2_body(ul_ref, ur_ref, dinv_ref, x_ref, sgw_ref, sgb_ref,
              a1w_ref, a1b_ref, a1a_ref, gw1_ref, gb1_ref, ga_ref,
              gw2_ref, gb2_ref, batch_ref, x1_ref, gate_ref, gmax_ref):
    i = pl.program_id(0)
    u = jnp.concatenate([ul_ref[...], ur_ref[...]], axis=1)
    h3 = u * dinv_ref[...]
    h = jnp.dot(h3, sgw_ref[...], preferred_element_type=F32) + sgb_ref[...]
    x1 = jnp.dot(h, a1w_ref[...], preferred_element_type=F32) + a1b_ref[...]
    a1 = a1a_ref[0, 0]
    x1 = jnp.where(x1 >= 0, x1, a1 * x1)
    x1_ref[...] = x1
    feats = jnp.concatenate([x_ref[...], x1], axis=1)
    g1 = jnp.dot(feats, gw1_ref[...], preferred_element_type=F32)
    g1 = g1 + gb1_ref[...]
    g1 = jnp.where(g1 >= 0, g1, ga_ref[...] * g1)
    gate = jnp.dot(g1, gw2_ref[...], preferred_element_type=F32)
    gate = gate + gb2_ref[0, 0]
    gate_ref[...] = gate

    onehot = batch_ref[...] == lax.broadcasted_iota(jnp.int32, (BLK, G), 1)
    gm = jnp.max(jnp.where(onehot, gate, jnp.float32(-3e38)), axis=0)[:, None]

    @pl.when(i == 0)
    def _():
        gmax_ref[...] = jnp.full((G, 1), -3e38, F32)

    gmax_ref[...] = jnp.maximum(gmax_ref[...], gm)


def _tc2(ul, ur, dinv, x, sgw, sgb, a1w, a1b, a1a, gw1, gb1, ga, gw2, gb2,
         batch2d):
    return pl.pallas_call(
        _tc2_body,
        grid=(N // BLK,),
        in_specs=[
            pl.BlockSpec((BLK, HALF), lambda i: (i, 0)),
            pl.BlockSpec((BLK, HALF), lambda i: (i, 0)),
            pl.BlockSpec((BLK, 1), lambda i: (i, 0)),
            pl.BlockSpec((BLK, HID), lambda i: (i, 0)),
            pl.BlockSpec((HID, 2 * HID), lambda i: (0, 0)),
            pl.BlockSpec((1, 2 * HID), lambda i: (0, 0)),
            pl.BlockSpec((2 * HID, HID), lambda i: (0, 0)),
            pl.BlockSpec((1, HID), lambda i: (0, 0)),
            pl.BlockSpec(memory_space=pltpu.SMEM),
            pl.BlockSpec((2 * HID, HID), lambda i: (0, 0)),
            pl.BlockSpec((1, HID), lambda i: (0, 0)),
            pl.BlockSpec((1, HID), lambda i: (0, 0)),
            pl.BlockSpec((HID, 1), lambda i: (0, 0)),
            pl.BlockSpec(memory_space=pltpu.SMEM),
            pl.BlockSpec((BLK, 1), lambda i: (i, 0)),
        ],
        out_specs=[
            pl.BlockSpec((BLK, HID), lambda i: (i, 0)),
            pl.BlockSpec((BLK, 1), lambda i: (i, 0)),
            pl.BlockSpec((G, 1), lambda i: (0, 0)),
        ],
        out_shape=[
            jax.ShapeDtypeStruct((N, HID), F32),
            jax.ShapeDtypeStruct((N, 1), F32),
            jax.ShapeDtypeStruct((G, 1), F32),
        ],
    )(ul, ur, dinv, x, sgw, sgb, a1w, a1b, a1a, gw1, gb1, ga, gw2, gb2,
      batch2d)


def _tc3_body(x_ref, x1_ref, gate_ref, gmax_ref, batch_ref, pw_ref, pb_ref,
              out_ref, racc, sacc):
    i = pl.program_id(0)

    @pl.when(i == 0)
    def _():
        racc[...] = jnp.zeros((G, 2 * HID), F32)
        sacc[...] = jnp.zeros((G, 1), F32)

    onehot = (batch_ref[...] ==
              lax.broadcasted_iota(jnp.int32, (BLK, G), 1)).astype(F32)
    gm = jnp.dot(onehot, gmax_ref[...], preferred_element_type=F32)
    e = jnp.exp(gate_ref[...] - gm)
    w = onehot * e
    feats = jnp.concatenate([x_ref[...], x1_ref[...]], axis=1)
    racc[...] += lax.dot_general(w, feats, (((0,), (0,)), ((), ())),
                                 preferred_element_type=F32)
    sacc[...] += jnp.sum(w, axis=0)[:, None]

    @pl.when(i == pl.num_programs(0) - 1)
    def _():
        reads = racc[...] / (sacc[...] + 1e-16)
        out_ref[...] = jnp.dot(reads, pw_ref[...],
                               preferred_element_type=F32) + pb_ref[...]


def _tc3(x, x1, gate, gmax, batch2d, pw, pb):
    return pl.pallas_call(
        _tc3_body,
        grid=(N // BLK,),
        in_specs=[
            pl.BlockSpec((BLK, HID), lambda i: (i, 0)),
            pl.BlockSpec((BLK, HID), lambda i: (i, 0)),
            pl.BlockSpec((BLK, 1), lambda i: (i, 0)),
            pl.BlockSpec((G, 1), lambda i: (0, 0)),
            pl.BlockSpec((BLK, 1), lambda i: (i, 0)),
            pl.BlockSpec((2 * HID, OUTC), lambda i: (0, 0)),
            pl.BlockSpec((1, OUTC), lambda i: (0, 0)),
        ],
        out_specs=pl.BlockSpec((G, OUTC), lambda i: (0, 0)),
        out_shape=jax.ShapeDtypeStruct((G, OUTC), F32),
        scratch_shapes=[
            pltpu.VMEM((G, 2 * HID), F32),
            pltpu.VMEM((G, 1), F32),
        ],
    )(x, x1, gate, gmax, batch2d, pw, pb)


OUTC = 128


def kernel(x, edge_index, batch, lin1_W, lin1_b, lin1_a, sg_W, sg_b,
           act1_W, act1_b, act1_a, gate_W1, gate_b1, gate_a, gate_W2,
           gate_b2, proj_W, proj_b):
    src = edge_index[0].reshape(NTILE, ESEG)
    dst = edge_index[1].reshape(NTILE, ESEG)
    padv = N + (jnp.arange(ESEGP - ESEG, dtype=jnp.int32) % (NPAD - N))
    pad2 = jnp.broadcast_to(padv, (NTILE, ESEGP - ESEG))
    srcp = jnp.concatenate([src, pad2], axis=1).reshape(NTILE, NCHUNK, CHUNK)
    dstp = jnp.concatenate([dst, pad2], axis=1).reshape(NTILE, NCHUNK, CHUNK)

    deg_k, prop_k = _sc_kernels()
    dega, degb = deg_k(dstp)

    g0l, g0r, invd, dinv = _tc1(
        x, lin1_W, lin1_b.reshape(1, HID), lin1_a.reshape(1, 1),
        dega.reshape(NPAD, 1), degb.reshape(NPAD, 1))

    ul, ur, _, _, _, _ = prop_k(g0l, g0r, srcp, dstp, invd.reshape(NPAD))

    batch2d = batch.reshape(N, 1)
    x1, gate, gmax = _tc2(
        ul, ur, dinv, x, sg_W, sg_b.reshape(1, 2 * HID),
        act1_W, act1_b.reshape(1, HID), act1_a.reshape(1, 1),
        gate_W1, gate_b1.reshape(1, HID), gate_a.reshape(1, HID),
        gate_W2, gate_b2.reshape(1, 1), batch2d)

    return _tc3(x, x1, gate, gmax, batch2d, proj_W, proj_b.reshape(1, OUTC))


# speedup vs baseline: 28.9880x; 1.0057x over previous
import functools

import jax
import jax.numpy as jnp
from jax import lax
from jax.experimental import pallas as pl
from jax.experimental.pallas import tpu as pltpu
from jax.experimental.pallas import tpu_sc as plsc

N = 10000
NPAD = 10240
HID = 128
HALF = 64
G = 8
NTILE = 16
RPT = NPAD // NTILE
E = 320000
ESEG = E // NTILE
CHUNK = 128
NCHUNK = 157
ESEGP = NCHUNK * CHUNK
BLK = 1000
F32 = jnp.float32

_MESH = dict(core_axis_name="c", subcore_axis_name="s", num_cores=2,
             num_subcores=NTILE)


def _deg_body(dst_hbm, dega_hbm, degb_hbm, dst_v, ones_v, zline_v, deg_sh,
              sem):
    c = lax.axis_index("c")
    s = lax.axis_index("s")
    pltpu.sync_copy(dst_hbm.at[s], dst_v)
    for j in range(CHUNK // 16):
        ones_v[pl.ds(j * 16, 16)] = jnp.full((16,), 1.0, F32)
    for j in range(RPT // 16):
        zline_v[pl.ds(j * 16, 16)] = jnp.zeros((16,), F32)
    pltpu.sync_copy(zline_v, deg_sh.at[pl.ds(s * RPT, RPT)])
    plsc.subcore_barrier()

    nk = 79 - c

    def body(i, _):
        k = 2 * i + c
        pltpu.async_copy(ones_v, deg_sh.at[dst_v.at[k]], sem, add=True).wait()
        return 0

    lax.fori_loop(0, nk, body, 0)
    plsc.subcore_barrier()

    @pl.when(c == 0)
    def _():
        pltpu.sync_copy(deg_sh.at[pl.ds(s * RPT, RPT)],
                        dega_hbm.at[pl.ds(s * RPT, RPT)])

    @pl.when(c == 1)
    def _():
        pltpu.sync_copy(deg_sh.at[pl.ds(s * RPT, RPT)],
                        degb_hbm.at[pl.ds(s * RPT, RPT)])


SUB = 80
NSUB = RPT // SUB
NBUF = 5
LEAD = 3


def _prop_body(g0l, g0r, srci, dsti, invd, ul, ur, t1l, t1r, t2l, t2r,
               esrc_v, dbuf_v, invd_v, slab_v, gbuf_v,
               gsem, ssem, dsem, acc_sh):
    c = lax.axis_index("c")
    s = lax.axis_index("s")
    row0 = s * RPT

    pltpu.sync_copy(srci.at[s], esrc_v)
    pltpu.sync_copy(invd.at[pl.ds(row0, RPT)], invd_v)

    for cval, g0 in ((0, g0l), (1, g0r)):
        @pl.when(c == cval)
        def _(g0=g0):
            for sub in range(NSUB):
                r0 = row0 + sub * SUB
                pltpu.sync_copy(g0.at[pl.ds(r0, SUB)], slab_v)
                pltpu.sync_copy(slab_v, acc_sh.at[pl.ds(r0, SUB)])
    plsc.subcore_barrier()

    srcs = ((g0l, t1l, t2l), (g0r, t1r, t2r))

    for r in range(3):
        for cval in (0, 1):
            gsrc = srcs[cval][r]

            @pl.when(c == cval)
            def _(gsrc=gsrc):
                def d_start(b, k):
                    pltpu.async_copy(dsti.at[s, k], dbuf_v.at[b],
                                     dsem.at[b])

                def d_wait(b, k):
                    pltpu.make_async_copy(dsti.at[s, k], dbuf_v.at[b],
                                          dsem.at[b]).wait()

                def g_start(b, k):
                    pltpu.async_copy(gsrc.at[esrc_v.at[k]], gbuf_v.at[b],
                                     gsem.at[b])

                def g_wait(b, k):
                    pltpu.make_async_copy(gsrc.at[esrc_v.at[k]],
                                          gbuf_v.at[b], gsem.at[b]).wait()

                def s_start(b, k):
                    pltpu.async_copy(gbuf_v.at[b], acc_sh.at[dbuf_v.at[b]],
                                     ssem.at[b], add=True)

                def s_wait(b, k):
                    pltpu.make_async_copy(gbuf_v.at[b],
                                          acc_sh.at[dbuf_v.at[b]],
                                          ssem.at[b]).wait()

                for p in range(LEAD):
                    d_start(p, p)
                    g_start(p, p)

                def ebody(k, _):
                    b = lax.rem(k, NBUF)
                    bl = lax.rem(k + LEAD, NBUF)

                    @pl.when(k >= NBUF - LEAD)
                    def _():
                        s_wait(bl, k - (NBUF - LEAD))

                    @pl.when(k + LEAD < NCHUNK)
                    def _():
                        d_start(bl, k + LEAD)
                        g_start(bl, k + LEAD)

                    g_wait(b, k)
                    d_wait(b, k)
                    s_start(b, k)
                    return 0

                lax.fori_loop(0, NCHUNK, ebody, 0)
                for t in range(NBUF - LEAD):
                    kk = NCHUNK - (NBUF - LEAD) + t
                    s_wait(kk % NBUF, kk)
        plsc.subcore_barrier()

        if r < 2:
            for sub in range(NSUB):
                r0 = row0 + sub * SUB
                pltpu.sync_copy(acc_sh.at[pl.ds(r0, SUB)], slab_v)

                def rbody(gi, _, sub=sub):
                    dv = invd_v[pl.ds(sub * SUB + gi * 16, 16)]
                    for rr in range(16):
                        d = dv[rr]
                        i = gi * 16 + rr
                        for j in range(HALF // 16):
                            sl = pl.ds(j * 16, 16)
                            slab_v[i, sl] = slab_v[i, sl] * d
                    return 0

                lax.fori_loop(0, SUB // 16, rbody, 0)
                pltpu.sync_copy(slab_v, acc_sh.at[pl.ds(r0, SUB)])
                for cval in (0, 1):
                    tnext = srcs[cval][r + 1]

                    @pl.when(c == cval)
                    def _(tnext=tnext):
                        pltpu.sync_copy(slab_v, tnext.at[pl.ds(r0, SUB)])
            plsc.subcore_barrier()
        else:
            for cval, uout in ((0, ul), (1, ur)):
                @pl.when(c == cval)
                def _(uout=uout):
                    pltpu.sync_copy(acc_sh.at[pl.ds(row0, RPT)],
                                    uout.at[pl.ds(row0, RPT)])


@functools.cache
def _sc_kernels():
    mesh = plsc.VectorSubcoreMesh(**_MESH)
    deg_k = pl.kernel(
        _deg_body,
        out_type=[jax.ShapeDtypeStruct((NPAD,), F32),
                  jax.ShapeDtypeStruct((NPAD,), F32)],
        mesh=mesh,
        scratch_types=[
            pltpu.VMEM((NCHUNK, CHUNK), jnp.int32),
            pltpu.VMEM((CHUNK,), F32),
            pltpu.VMEM((RPT,), F32),
            pltpu.VMEM_SHARED((NPAD,), F32),
            pltpu.SemaphoreType.DMA,
        ],
    )
    prop_k = pl.kernel(
        _prop_body,
        out_type=[jax.ShapeDtypeStruct((NPAD, HALF), F32)] * 6,
        mesh=mesh,
        scratch_types=[
            pltpu.VMEM((NCHUNK, CHUNK), jnp.int32),
            pltpu.VMEM((NBUF, CHUNK), jnp.int32),
            pltpu.VMEM((RPT,), F32),
            pltpu.VMEM((SUB, HALF), F32),
            pltpu.VMEM((NBUF, CHUNK, HALF), F32),
            pltpu.SemaphoreType.DMA((NBUF,)),
            pltpu.SemaphoreType.DMA((NBUF,)),
            pltpu.SemaphoreType.DMA((NBUF,)),
            pltpu.VMEM_SHARED((NPAD, HALF), F32),
        ],
        compiler_params=pltpu.CompilerParams(use_tc_tiling_on_sc=False),
    )
    return deg_k, prop_k


def _tc1_body(x_ref, w_ref, b_ref, a_ref, dega_ref, degb_ref,
              g0l_ref, g0r_ref, invd_ref, dinv_ref):
    h = jnp.dot(x_ref[...], w_ref[...], preferred_element_type=F32)
    h = h + b_ref[...]
    a = a_ref[0, 0]
    h = jnp.where(h >= 0, h, a * h)
    deg = dega_ref[...] + degb_ref[...] + 1.0
    dinv = lax.rsqrt(deg)
    g0 = h * dinv
    g0l_ref[...] = g0[:, :HALF]
    g0r_ref[...] = g0[:, HALF:]
    invd_ref[...] = 1.0 / deg
    dinv_ref[...] = dinv


def _tc1(x, w, b, a, dega, degb):
    return pl.pallas_call(
        _tc1_body,
        grid=(N // BLK,),
        in_specs=[
            pl.BlockSpec((BLK, HID), lambda i: (i, 0)),
            pl.BlockSpec((HID, HID), lambda i: (0, 0)),
            pl.BlockSpec((1, HID), lambda i: (0, 0)),
            pl.BlockSpec(memory_space=pltpu.SMEM),
            pl.BlockSpec((BLK, 1), lambda i: (i, 0)),
            pl.BlockSpec((BLK, 1), lambda i: (i, 0)),
        ],
        out_specs=[
            pl.BlockSpec((BLK, HALF), lambda i: (i, 0)),
            pl.BlockSpec((BLK, HALF), lambda i: (i, 0)),
            pl.BlockSpec((BLK, 1), lambda i: (i, 0)),
            pl.BlockSpec((BLK, 1), lambda i: (i, 0)),
        ],
        out_shape=[
            jax.ShapeDtypeStruct((NPAD, HALF), F32),
            jax.ShapeDtypeStruct((NPAD, HALF), F32),
            jax.ShapeDtypeStruct((NPAD, 1), F32),
            jax.ShapeDtypeStruct((NPAD, 1), F32),
        ],
    )(x, w, b, a, dega, degb)


def _tc23_body(ul_ref, ur_ref, dinv_ref, x_ref, sgw_ref, sgb_ref,
               a1w_ref, a1b_ref, a1a_ref, gw1_ref, gb1_ref, ga_ref,
               gw2_ref, gb2_ref, batch_ref, pw_ref, pb_ref,
               out_ref, x1s, gates, gmaxs, racc, sacc):
    p = pl.program_id(0)
    i = pl.program_id(1)
    onehot_b = batch_ref[...] == lax.broadcasted_iota(jnp.int32, (BLK, G), 1)

    @pl.when(p == 0)
    def _():
        u = jnp.concatenate([ul_ref[...], ur_ref[...]], axis=1)
        h3 = u * dinv_ref[...]
        h = jnp.dot(h3, sgw_ref[...], preferred_element_type=F32)
        h = h + sgb_ref[...]
        x1 = jnp.dot(h, a1w_ref[...], preferred_element_type=F32)
        x1 = x1 + a1b_ref[...]
        a1 = a1a_ref[0, 0]
        x1 = jnp.where(x1 >= 0, x1, a1 * x1)
        x1s[pl.ds(i * BLK, BLK), :] = x1
        feats = jnp.concatenate([x_ref[...], x1], axis=1)
        g1 = jnp.dot(feats, gw1_ref[...], preferred_element_type=F32)
        g1 = g1 + gb1_ref[...]
        g1 = jnp.where(g1 >= 0, g1, ga_ref[...] * g1)
        gate = jnp.dot(g1, gw2_ref[...], preferred_element_type=F32)
        gate = gate + gb2_ref[0, 0]
        gates[pl.ds(i * BLK, BLK), :] = gate

        gm = jnp.max(jnp.where(onehot_b, gate, jnp.float32(-3e38)),
                     axis=0)[:, None]

        @pl.when(i == 0)
        def _():
            gmaxs[...] = jnp.full((G, 1), -3e38, F32)

        gmaxs[...] = jnp.maximum(gmaxs[...], gm)

    @pl.when(p == 1)
    def _():
        @pl.when(i == 0)
        def _():
            racc[...] = jnp.zeros((G, 2 * HID), F32)
            sacc[...] = jnp.zeros((G, 1), F32)

        onehot = onehot_b.astype(F32)
        gm = jnp.dot(onehot, gmaxs[...], preferred_element_type=F32)
        e = jnp.exp(gates[pl.ds(i * BLK, BLK), :] - gm)
        w = onehot * e
        feats = jnp.concatenate([x_ref[...], x1s[pl.ds(i * BLK, BLK), :]],
                                axis=1)
        racc[...] += lax.dot_general(w, feats, (((0,), (0,)), ((), ())),
                                     preferred_element_type=F32)
        sacc[...] += jnp.sum(w, axis=0)[:, None]

        @pl.when(i == pl.num_programs(1) - 1)
        def _():
            reads = racc[...] / (sacc[...] + 1e-16)
            out_ref[...] = jnp.dot(reads, pw_ref[...],
                                   preferred_element_type=F32) + pb_ref[...]


def _tc23(ul, ur, dinv, x, sgw, sgb, a1w, a1b, a1a, gw1, gb1, ga, gw2, gb2,
          batch2d, pw, pb):
    return pl.pallas_call(
        _tc23_body,
        grid=(2, N // BLK),
        in_specs=[
            pl.BlockSpec((BLK, HALF), lambda p, i: (i, 0)),
            pl.BlockSpec((BLK, HALF), lambda p, i: (i, 0)),
            pl.BlockSpec((BLK, 1), lambda p, i: (i, 0)),
            pl.BlockSpec((BLK, HID), lambda p, i: (i, 0)),
            pl.BlockSpec((HID, 2 * HID), lambda p, i: (0, 0)),
            pl.BlockSpec((1, 2 * HID), lambda p, i: (0, 0)),
            pl.BlockSpec((2 * HID, HID), lambda p, i: (0, 0)),
            pl.BlockSpec((1, HID), lambda p, i: (0, 0)),
            pl.BlockSpec(memory_space=pltpu.SMEM),
            pl.BlockSpec((2 * HID, HID), lambda p, i: (0, 0)),
            pl.BlockSpec((1, HID), lambda p, i: (0, 0)),
            pl.BlockSpec((1, HID), lambda p, i: (0, 0)),
            pl.BlockSpec((HID, 1), lambda p, i: (0, 0)),
            pl.BlockSpec(memory_space=pltpu.SMEM),
            pl.BlockSpec((BLK, 1), lambda p, i: (i, 0)),
            pl.BlockSpec((2 * HID, OUTC), lambda p, i: (0, 0)),
            pl.BlockSpec((1, OUTC), lambda p, i: (0, 0)),
        ],
        out_specs=pl.BlockSpec((G, OUTC), lambda p, i: (0, 0)),
        out_shape=jax.ShapeDtypeStruct((G, OUTC), F32),
        scratch_shapes=[
            pltpu.VMEM((N, HID), F32),
            pltpu.VMEM((N, 1), F32),
            pltpu.VMEM((G, 1), F32),
            pltpu.VMEM((G, 2 * HID), F32),
            pltpu.VMEM((G, 1), F32),
        ],
    )(ul, ur, dinv, x, sgw, sgb, a1w, a1b, a1a, gw1, gb1, ga, gw2, gb2,
      batch2d, pw, pb)


OUTC = 128


def kernel(x, edge_index, batch, lin1_W, lin1_b, lin1_a, sg_W, sg_b,
           act1_W, act1_b, act1_a, gate_W1, gate_b1, gate_a, gate_W2,
           gate_b2, proj_W, proj_b):
    src = edge_index[0].reshape(NTILE, ESEG)
    dst = edge_index[1].reshape(NTILE, ESEG)
    padv = N + (jnp.arange(ESEGP - ESEG, dtype=jnp.int32) % (NPAD - N))
    pad2 = jnp.broadcast_to(padv, (NTILE, ESEGP - ESEG))
    srcp = jnp.concatenate([src, pad2], axis=1).reshape(NTILE, NCHUNK, CHUNK)
    dstp = jnp.concatenate([dst, pad2], axis=1).reshape(NTILE, NCHUNK, CHUNK)

    deg_k, prop_k = _sc_kernels()
    dega, degb = deg_k(dstp)

    g0l, g0r, invd, dinv = _tc1(
        x, lin1_W, lin1_b.reshape(1, HID), lin1_a.reshape(1, 1),
        dega.reshape(NPAD, 1), degb.reshape(NPAD, 1))

    ul, ur, _, _, _, _ = prop_k(g0l, g0r, srcp, dstp, invd.reshape(NPAD))

    batch2d = batch.reshape(N, 1)
    return _tc23(
        ul, ur, dinv, x, sg_W, sg_b.reshape(1, 2 * HID),
        act1_W, act1_b.reshape(1, HID), act1_a.reshape(1, 1),
        gate_W1, gate_b1.reshape(1, HID), gate_a.reshape(1, HID),
        gate_W2, gate_b2.reshape(1, 1), batch2d, proj_W,
        proj_b.reshape(1, OUTC))
